# Initial kernel scaffold; baseline (speedup 1.0000x reference)
import jax
import jax.numpy as jnp
from jax.experimental import pallas as pl


def kernel(map_x, edges, pl_rel, dst_x, params):
    raise NotImplementedError("write your pallas kernel here")


# speedup vs baseline: 15.5576x; 15.5576x over previous
import functools

import jax
import jax.numpy as jnp
from jax import lax
from jax.experimental import pallas as pl
from jax.experimental.pallas import tpu as pltpu
from jax.experimental.pallas import tpu_sc as plsc

HID = 128
NFREQ = 64
NHEAD = 8
HDIM = HID // NHEAD
N_PL = 10000
N_DST = 64 * 6 * 60
E = 368640
LNEPS = 1e-5
SCALE = HDIM ** -0.5

C_EDGE = 128
T_EDGE = 1024
T_DST = 384
HN = N_DST // 2
NR = 11648
STRIPE = NR // 16


def _ln(x):
    mu = x.mean(-1, keepdims=True)
    var = ((x - mu) ** 2).mean(-1, keepdims=True)
    return (x - mu) / jnp.sqrt(var + LNEPS)


def _src_prep_body(mx_ref, wk_ref, wv_ref, kv_ref):
    xs = _ln(mx_ref[...])
    kv_ref[...] = jnp.concatenate([xs @ wk_ref[...], xs @ wv_ref[...]], axis=1)


def _dst_prep_body(d_ref, wq_ref, bq_ref, q_ref, xd_ref):
    xd = _ln(d_ref[...])
    q_ref[...] = xd @ wq_ref[...] + bq_ref[...]
    xd_ref[...] = xd


def _node_prep(map_x, d, p):
    kv = pl.pallas_call(
        _src_prep_body,
        grid=(10,),
        in_specs=[
            pl.BlockSpec((N_PL // 10, HID), lambda i: (i, 0)),
            pl.BlockSpec((HID, HID), lambda i: (0, 0)),
            pl.BlockSpec((HID, HID), lambda i: (0, 0)),
        ],
        out_specs=pl.BlockSpec((N_PL // 10, 2 * HID), lambda i: (i, 0)),
        out_shape=jax.ShapeDtypeStruct((N_PL, 2 * HID), jnp.float32),
    )(map_x, p['wk'], p['wv'])
    q, xd = pl.pallas_call(
        _dst_prep_body,
        grid=(N_DST // T_DST,),
        in_specs=[
            pl.BlockSpec((T_DST, HID), lambda i: (i, 0)),
            pl.BlockSpec((HID, HID), lambda i: (0, 0)),
            pl.BlockSpec((1, HID), lambda i: (0, 0)),
        ],
        out_specs=[
            pl.BlockSpec((T_DST, HID), lambda i: (i, 0)),
            pl.BlockSpec((T_DST, HID), lambda i: (i, 0)),
        ],
        out_shape=[
            jax.ShapeDtypeStruct((N_DST, HID), jnp.float32),
            jax.ShapeDtypeStruct((N_DST, HID), jnp.float32),
        ],
    )(d, p['wq'], p['bq'].reshape(1, HID))
    return kv, q, xd


def _make_gather():
    mesh = plsc.VectorSubcoreMesh(core_axis_name="c", subcore_axis_name="s")

    @functools.partial(
        pl.kernel,
        mesh=mesh,
        out_type=[
            jax.ShapeDtypeStruct((E, 2 * HID), jnp.float32),
            jax.ShapeDtypeStruct((E, HID), jnp.float32),
        ],
        scratch_types=[
            pltpu.VMEM((C_EDGE,), jnp.int32),
            pltpu.VMEM((C_EDGE,), jnp.int32),
            pltpu.VMEM((C_EDGE, 2 * HID), jnp.float32),
            pltpu.VMEM((C_EDGE, HID), jnp.float32),
            pltpu.SemaphoreType.DMA,
            pltpu.SemaphoreType.DMA,
        ],
    )
    def gath(kv_hbm, q_hbm, src_hbm, dst_hbm, ksvs_hbm, qd_hbm,
             sidx, didx, kvbuf, qbuf, sem1, sem2):
        wid = lax.axis_index("s") * 2 + lax.axis_index("c")
        per = E // 32

        def body(i, carry):
            base = wid * per + i * C_EDGE
            pltpu.sync_copy(src_hbm.at[pl.ds(base, C_EDGE)], sidx)
            pltpu.sync_copy(dst_hbm.at[pl.ds(base, C_EDGE)], didx)
            cp1 = pltpu.async_copy(kv_hbm.at[sidx], kvbuf, sem1)
            cp2 = pltpu.async_copy(q_hbm.at[didx], qbuf, sem2)
            cp1.wait()
            cp2.wait()
            pltpu.sync_copy(kvbuf, ksvs_hbm.at[pl.ds(base, C_EDGE)])
            pltpu.sync_copy(qbuf, qd_hbm.at[pl.ds(base, C_EDGE)])
            return carry

        lax.fori_loop(0, per // C_EDGE, body, 0)

    return gath


def _edge_body(rel_ref, ksvs_ref, qd_ref, dst_ref, freqs_ref, w1_ref, b1_ref,
               w2_ref, wo_ref, bo_ref, wkr_ref, wvr_ref, u_ref, wb_ref,
               didx_ref):
    rel = rel_ref[...]
    acc = jnp.zeros((rel.shape[0], HID), jnp.float32)
    freqs = freqs_ref[...]
    for i in range(3):
        xi = rel[:, i:i + 1]
        fi = xi * (freqs[i][None, :] * (2.0 * jnp.pi))
        w1 = w1_ref[...][i]
        hi = (jnp.cos(fi) @ w1[:NFREQ]
              + jnp.sin(fi) @ w1[NFREQ:2 * NFREQ]
              + xi * w1[2 * NFREQ][None, :]
              + b1_ref[...][i][None, :])
        hi = jnp.maximum(_ln(hi), 0.0)
        acc = acc + hi @ w2_ref[...][i]
    relh = jnp.maximum(_ln(acc), 0.0) @ wo_ref[...] + bo_ref[...]
    r = _ln(relh)
    kr = r @ wkr_ref[...]
    vr = r @ wvr_ref[...]
    ksvs = ksvs_ref[...]
    k_e = ksvs[:, :HID] + kr
    v_e = ksvs[:, HID:] + vr
    prod = qd_ref[...] * k_e
    hsel = (lax.broadcasted_iota(jnp.int32, (HID, NHEAD), 0) // HDIM
            == lax.broadcasted_iota(jnp.int32, (HID, NHEAD), 1))
    bd = hsel.astype(jnp.float32)
    sim = (prod @ bd) * SCALE
    w128 = jnp.exp(sim) @ bd.T
    u_ref[...] = w128 * v_e
    wb_ref[...] = w128
    dstv = dst_ref[0]
    d0 = jnp.where(dstv < HN, dstv, HN)
    d1 = jnp.where(dstv >= HN, dstv - HN, HN)
    didx_ref[...] = jnp.concatenate([d0, d1], axis=0)


def _edge_call(pl_rel, ksvs, qd, dst3, p):
    full = lambda shape: pl.BlockSpec(shape, lambda i: tuple(0 for _ in shape))
    return pl.pallas_call(
        _edge_body,
        grid=(E // T_EDGE,),
        in_specs=[
            pl.BlockSpec((T_EDGE, 3), lambda i: (i, 0)),
            pl.BlockSpec((T_EDGE, 2 * HID), lambda i: (i, 0)),
            pl.BlockSpec((T_EDGE, HID), lambda i: (i, 0)),
            pl.BlockSpec((1, 1, T_EDGE), lambda i: (i, 0, 0)),
            full((3, NFREQ)),
            full((3, 2 * NFREQ + 1, HID)),
            full((3, HID)),
            full((3, HID, HID)),
            full((HID, HID)),
            full((1, HID)),
            full((HID, HID)),
            full((HID, HID)),
        ],
        out_specs=[
            pl.BlockSpec((T_EDGE, HID), lambda i: (i, 0)),
            pl.BlockSpec((T_EDGE, HID), lambda i: (i, 0)),
            pl.BlockSpec((2, T_EDGE), lambda i: (0, i)),
        ],
        out_shape=[
            jax.ShapeDtypeStruct((E, HID), jnp.float32),
            jax.ShapeDtypeStruct((E, HID), jnp.float32),
            jax.ShapeDtypeStruct((2, E), jnp.int32),
        ],
    )(pl_rel, ksvs, qd, dst3, p['fe_freqs'], p['fe_w1'], p['fe_b1'],
      p['fe_w2'], p['fe_wo'], p['fe_bo'].reshape(1, HID), p['wkr'], p['wvr'])


def _make_scatter():
    mesh = plsc.VectorSubcoreMesh(core_axis_name="c", subcore_axis_name="s")

    @functools.partial(
        pl.kernel,
        mesh=mesh,
        out_type=jax.ShapeDtypeStruct((2, NR, HID), jnp.float32),
        scratch_types=[
            pltpu.VMEM((C_EDGE,), jnp.int32),
            pltpu.VMEM((C_EDGE, HID), jnp.float32),
            pltpu.VMEM_SHARED((NR, HID), jnp.float32),
        ],
    )
    def scat(vals_hbm, idx_hbm, z_hbm, out_hbm, didx, vbuf, acc):
        c = lax.axis_index("c")
        s = lax.axis_index("s")
        pltpu.sync_copy(z_hbm, acc.at[pl.ds(s * STRIPE, STRIPE)])
        plsc.subcore_barrier()
        per = E // 16

        def body(i, carry):
            base = s * per + i * C_EDGE
            pltpu.sync_copy(idx_hbm.at[pl.ds(c * E + base, C_EDGE)], didx)
            pltpu.sync_copy(vals_hbm.at[pl.ds(base, C_EDGE)], vbuf)
            pltpu.sync_copy(vbuf, acc.at[didx], add=True)
            return carry

        lax.fori_loop(0, per // C_EDGE, body, 0)
        plsc.subcore_barrier()
        pltpu.sync_copy(acc.at[pl.ds(s * STRIPE, STRIPE)],
                        out_hbm.at[c, pl.ds(s * STRIPE, STRIPE)])

    return scat


def _final_body(agg_ref, den_ref, xd_ref, draw_ref,
                wg_ref, bg_ref, ws_ref, bs_ref, wo_ref, bo_ref, w1_ref, b1_ref,
                w2_ref, b2_ref, out_ref):
    agg_u = agg_ref[0]
    den128 = den_ref[0]
    agg = agg_u / (den128 + 1e-9)
    xd = xd_ref[...]
    wg = wg_ref[...]
    g = jax.nn.sigmoid(agg @ wg[:HID] + xd @ wg[HID:] + bg_ref[...])
    sv = xd @ ws_ref[...] + bs_ref[...]
    outv = agg + g * (sv - agg)
    x = draw_ref[...] + outv @ wo_ref[...] + bo_ref[...]
    h = _ln(x)
    out_ref[...] = (x + jnp.maximum(h @ w1_ref[...] + b1_ref[...], 0.0)
                    @ w2_ref[...] + b2_ref[...])


def _final_call(agg2, den2, xd, d, p):
    full = lambda shape: pl.BlockSpec(shape, lambda i: tuple(0 for _ in shape))
    row = lambda w: pl.BlockSpec((T_DST, w), lambda i: (i, 0))
    nhalf = HN // T_DST
    acc_spec = pl.BlockSpec((1, T_DST, HID),
                            lambda i: (i // nhalf, i % nhalf, 0))
    return pl.pallas_call(
        _final_body,
        grid=(N_DST // T_DST,),
        in_specs=[
            acc_spec, acc_spec,
            row(HID), row(HID),
            full((2 * HID, HID)), full((1, HID)),
            full((HID, HID)), full((1, HID)),
            full((HID, HID)), full((1, HID)),
            full((HID, 4 * HID)), full((1, 4 * HID)),
            full((4 * HID, HID)), full((1, HID)),
        ],
        out_specs=pl.BlockSpec((T_DST, HID), lambda i: (i, 0)),
        out_shape=jax.ShapeDtypeStruct((N_DST, HID), jnp.float32),
    )(agg2, den2, xd, d,
      p['wg'], p['bg'].reshape(1, HID),
      p['ws'], p['bs'].reshape(1, HID),
      p['wo'], p['bo'].reshape(1, HID),
      p['w1'], p['b1'].reshape(1, 4 * HID),
      p['w2'], p['b2'].reshape(1, HID))


def kernel(map_x, edges, pl_rel, dst_x, params):
    p = params
    src = edges[0]
    dst = edges[1]
    d = dst_x.reshape(N_DST, HID)

    kv, q, xd = _node_prep(map_x, d, p)
    ksvs, qd = _make_gather()(kv, q, src, dst)
    dst3 = dst.reshape(E // T_EDGE, 1, T_EDGE)
    u, wb, didx2 = _edge_call(pl_rel, ksvs, qd, dst3, p)
    idx_cat = didx2.reshape(2 * E)
    z = jnp.zeros((STRIPE, HID), jnp.float32)
    scat = _make_scatter()
    agg2 = scat(u, idx_cat, z)
    den2 = scat(wb, idx_cat, z)
    y = _final_call(agg2, den2, xd, d, p)
    return y.reshape(dst_x.shape)


# speedup vs baseline: 23.4938x; 1.5101x over previous
import functools

import jax
import jax.numpy as jnp
from jax import lax
from jax.experimental import pallas as pl
from jax.experimental.pallas import tpu as pltpu
from jax.experimental.pallas import tpu_sc as plsc

HID = 128
NFREQ = 64
NHEAD = 8
HDIM = HID // NHEAD
N_PL = 10000
N_DST = 64 * 6 * 60
E = 368640
LNEPS = 1e-5
SCALE = HDIM ** -0.5

C_EDGE = 128
T_EDGE = 1024
T_DST = 384
HN = N_DST // 2
NR = 11648
STRIPE = NR // 16


def _ln(x):
    mu = x.mean(-1, keepdims=True)
    var = ((x - mu) ** 2).mean(-1, keepdims=True)
    return (x - mu) * lax.rsqrt(var + LNEPS)


_COSC = (0.9999999922855516, -19.739205552336067, 64.939172135788,
         -85.45116383102747, 60.17621268245674, -26.000455681225553,
         6.575502264028825)
_SINC = (6.283185303890264, -41.34170085524102, 81.60515475045466,
         -76.70345307237866, 42.029594285416415, -14.913887579943939,
         3.258156077668993)


def _poly(s, coeffs):
    acc = jnp.full_like(s, coeffs[-1])
    for c in coeffs[-2::-1]:
        acc = acc * s + c
    return acc


def _src_prep_body(mx_ref, wk_ref, wv_ref, kv_ref):
    xs = _ln(mx_ref[...])
    kv_ref[...] = jnp.concatenate([xs @ wk_ref[...], xs @ wv_ref[...]], axis=1)


def _dst_prep_body(d_ref, wq_ref, bq_ref, q_ref, xd_ref):
    xd = _ln(d_ref[...])
    q_ref[...] = xd @ wq_ref[...] + bq_ref[...]
    xd_ref[...] = xd


def _node_prep(map_x, d, p):
    kv = pl.pallas_call(
        _src_prep_body,
        grid=(10,),
        in_specs=[
            pl.BlockSpec((N_PL // 10, HID), lambda i: (i, 0)),
            pl.BlockSpec((HID, HID), lambda i: (0, 0)),
            pl.BlockSpec((HID, HID), lambda i: (0, 0)),
        ],
        out_specs=pl.BlockSpec((N_PL // 10, 2 * HID), lambda i: (i, 0)),
        out_shape=jax.ShapeDtypeStruct((N_PL, 2 * HID), jnp.float32),
    )(map_x, p['wk'], p['wv'])
    q, xd = pl.pallas_call(
        _dst_prep_body,
        grid=(N_DST // T_DST,),
        in_specs=[
            pl.BlockSpec((T_DST, HID), lambda i: (i, 0)),
            pl.BlockSpec((HID, HID), lambda i: (0, 0)),
            pl.BlockSpec((1, HID), lambda i: (0, 0)),
        ],
        out_specs=[
            pl.BlockSpec((T_DST, HID), lambda i: (i, 0)),
            pl.BlockSpec((T_DST, HID), lambda i: (i, 0)),
        ],
        out_shape=[
            jax.ShapeDtypeStruct((N_DST, HID), jnp.float32),
            jax.ShapeDtypeStruct((N_DST, HID), jnp.float32),
        ],
    )(d, p['wq'], p['bq'].reshape(1, HID))
    return kv, q, xd


def _make_gather():
    mesh = plsc.VectorSubcoreMesh(core_axis_name="c", subcore_axis_name="s")

    @functools.partial(
        pl.kernel,
        mesh=mesh,
        out_type=[
            jax.ShapeDtypeStruct((E, 2 * HID), jnp.float32),
            jax.ShapeDtypeStruct((E, HID), jnp.float32),
        ],
        scratch_types=[
            pltpu.VMEM((C_EDGE,), jnp.int32),
            pltpu.VMEM((C_EDGE,), jnp.int32),
            pltpu.VMEM((C_EDGE, 2 * HID), jnp.float32),
            pltpu.VMEM((C_EDGE, HID), jnp.float32),
            pltpu.SemaphoreType.DMA,
            pltpu.SemaphoreType.DMA,
        ],
    )
    def gath(kv_hbm, q_hbm, src_hbm, dst_hbm, ksvs_hbm, qd_hbm,
             sidx, didx, kvbuf, qbuf, sem1, sem2):
        wid = lax.axis_index("s") * 2 + lax.axis_index("c")
        per = E // 32

        def body(i, carry):
            base = wid * per + i * C_EDGE
            pltpu.sync_copy(src_hbm.at[pl.ds(base, C_EDGE)], sidx)
            pltpu.sync_copy(dst_hbm.at[pl.ds(base, C_EDGE)], didx)
            cp1 = pltpu.async_copy(kv_hbm.at[sidx], kvbuf, sem1)
            cp2 = pltpu.async_copy(q_hbm.at[didx], qbuf, sem2)
            cp1.wait()
            cp2.wait()
            pltpu.sync_copy(kvbuf, ksvs_hbm.at[pl.ds(base, C_EDGE)])
            pltpu.sync_copy(qbuf, qd_hbm.at[pl.ds(base, C_EDGE)])
            return carry

        lax.fori_loop(0, per // C_EDGE, body, 0)

    return gath


def _edge_body(rel_ref, ksvs_ref, qd_ref, dst_ref, freqs_ref, w1_ref, b1_ref,
               w2_ref, wo_ref, bo_ref, wkr_ref, wvr_ref, u_ref, wb_ref,
               didx_ref):
    rel = rel_ref[...]
    acc = jnp.zeros((rel.shape[0], HID), jnp.float32)
    freqs = freqs_ref[...]
    for i in range(3):
        xi = rel[:, i:i + 1]
        z = xi * freqs[i][None, :]
        r = z - jnp.round(z)
        sq = r * r
        cosf = _poly(sq, _COSC)
        sinf = r * _poly(sq, _SINC)
        w1 = w1_ref[...][i]
        hi = (cosf @ w1[:NFREQ]
              + sinf @ w1[NFREQ:2 * NFREQ]
              + xi * w1[2 * NFREQ][None, :]
              + b1_ref[...][i][None, :])
        hi = jnp.maximum(_ln(hi), 0.0)
        acc = acc + hi @ w2_ref[...][i]
    relh = jnp.maximum(_ln(acc), 0.0) @ wo_ref[...] + bo_ref[...]
    r = _ln(relh)
    kr = r @ wkr_ref[...]
    vr = r @ wvr_ref[...]
    ksvs = ksvs_ref[...]
    k_e = ksvs[:, :HID] + kr
    v_e = ksvs[:, HID:] + vr
    prod = qd_ref[...] * k_e
    hsel = (lax.broadcasted_iota(jnp.int32, (HID, NHEAD), 0) // HDIM
            == lax.broadcasted_iota(jnp.int32, (HID, NHEAD), 1))
    bd = hsel.astype(jnp.float32)
    sim = (prod @ bd) * SCALE
    w128 = jnp.exp(sim) @ bd.T
    u_ref[...] = w128 * v_e
    wb_ref[...] = w128
    dstv = dst_ref[0]
    d0 = jnp.where(dstv < HN, dstv, HN)
    d1 = jnp.where(dstv >= HN, dstv - HN, HN)
    didx_ref[...] = jnp.concatenate([d0, d1], axis=0)


def _edge_call(pl_rel, ksvs, qd, dst3, p):
    full = lambda shape: pl.BlockSpec(shape, lambda i: tuple(0 for _ in shape))
    return pl.pallas_call(
        _edge_body,
        grid=(E // T_EDGE,),
        in_specs=[
            pl.BlockSpec((T_EDGE, 3), lambda i: (i, 0)),
            pl.BlockSpec((T_EDGE, 2 * HID), lambda i: (i, 0)),
            pl.BlockSpec((T_EDGE, HID), lambda i: (i, 0)),
            pl.BlockSpec((1, 1, T_EDGE), lambda i: (i, 0, 0)),
            full((3, NFREQ)),
            full((3, 2 * NFREQ + 1, HID)),
            full((3, HID)),
            full((3, HID, HID)),
            full((HID, HID)),
            full((1, HID)),
            full((HID, HID)),
            full((HID, HID)),
        ],
        out_specs=[
            pl.BlockSpec((T_EDGE, HID), lambda i: (i, 0)),
            pl.BlockSpec((T_EDGE, HID), lambda i: (i, 0)),
            pl.BlockSpec((2, T_EDGE), lambda i: (0, i)),
        ],
        out_shape=[
            jax.ShapeDtypeStruct((E, HID), jnp.float32),
            jax.ShapeDtypeStruct((E, HID), jnp.float32),
            jax.ShapeDtypeStruct((2, E), jnp.int32),
        ],
    )(pl_rel, ksvs, qd, dst3, p['fe_freqs'], p['fe_w1'], p['fe_b1'],
      p['fe_w2'], p['fe_wo'], p['fe_bo'].reshape(1, HID), p['wkr'], p['wvr'])


def _make_scatter():
    mesh = plsc.VectorSubcoreMesh(core_axis_name="c", subcore_axis_name="s")

    @functools.partial(
        pl.kernel,
        mesh=mesh,
        out_type=jax.ShapeDtypeStruct((2, NR, HID), jnp.float32),
        scratch_types=[
            pltpu.VMEM((C_EDGE,), jnp.int32),
            pltpu.VMEM((C_EDGE, HID), jnp.float32),
            pltpu.VMEM_SHARED((NR, HID), jnp.float32),
        ],
    )
    def scat(vals_hbm, idx_hbm, z_hbm, out_hbm, didx, vbuf, acc):
        c = lax.axis_index("c")
        s = lax.axis_index("s")
        pltpu.sync_copy(z_hbm, acc.at[pl.ds(s * STRIPE, STRIPE)])
        plsc.subcore_barrier()
        per = E // 16

        def body(i, carry):
            base = s * per + i * C_EDGE
            pltpu.sync_copy(idx_hbm.at[pl.ds(c * E + base, C_EDGE)], didx)
            pltpu.sync_copy(vals_hbm.at[pl.ds(base, C_EDGE)], vbuf)
            pltpu.sync_copy(vbuf, acc.at[didx], add=True)
            return carry

        lax.fori_loop(0, per // C_EDGE, body, 0)
        plsc.subcore_barrier()
        pltpu.sync_copy(acc.at[pl.ds(s * STRIPE, STRIPE)],
                        out_hbm.at[c, pl.ds(s * STRIPE, STRIPE)])

    return scat


def _final_body(agg_ref, den_ref, xd_ref, draw_ref,
                wg_ref, bg_ref, ws_ref, bs_ref, wo_ref, bo_ref, w1_ref, b1_ref,
                w2_ref, b2_ref, out_ref):
    agg_u = agg_ref[0]
    den128 = den_ref[0]
    agg = agg_u / (den128 + 1e-9)
    xd = xd_ref[...]
    wg = wg_ref[...]
    g = jax.nn.sigmoid(agg @ wg[:HID] + xd @ wg[HID:] + bg_ref[...])
    sv = xd @ ws_ref[...] + bs_ref[...]
    outv = agg + g * (sv - agg)
    x = draw_ref[...] + outv @ wo_ref[...] + bo_ref[...]
    h = _ln(x)
    out_ref[...] = (x + jnp.maximum(h @ w1_ref[...] + b1_ref[...], 0.0)
                    @ w2_ref[...] + b2_ref[...])


def _final_call(agg2, den2, xd, d, p):
    full = lambda shape: pl.BlockSpec(shape, lambda i: tuple(0 for _ in shape))
    row = lambda w: pl.BlockSpec((T_DST, w), lambda i: (i, 0))
    nhalf = HN // T_DST
    acc_spec = pl.BlockSpec((1, T_DST, HID),
                            lambda i: (i // nhalf, i % nhalf, 0))
    return pl.pallas_call(
        _final_body,
        grid=(N_DST // T_DST,),
        in_specs=[
            acc_spec, acc_spec,
            row(HID), row(HID),
            full((2 * HID, HID)), full((1, HID)),
            full((HID, HID)), full((1, HID)),
            full((HID, HID)), full((1, HID)),
            full((HID, 4 * HID)), full((1, 4 * HID)),
            full((4 * HID, HID)), full((1, HID)),
        ],
        out_specs=pl.BlockSpec((T_DST, HID), lambda i: (i, 0)),
        out_shape=jax.ShapeDtypeStruct((N_DST, HID), jnp.float32),
    )(agg2, den2, xd, d,
      p['wg'], p['bg'].reshape(1, HID),
      p['ws'], p['bs'].reshape(1, HID),
      p['wo'], p['bo'].reshape(1, HID),
      p['w1'], p['b1'].reshape(1, 4 * HID),
      p['w2'], p['b2'].reshape(1, HID))


def kernel(map_x, edges, pl_rel, dst_x, params):
    p = params
    src = edges[0]
    dst = edges[1]
    d = dst_x.reshape(N_DST, HID)

    kv, q, xd = _node_prep(map_x, d, p)
    ksvs, qd = _make_gather()(kv, q, src, dst)
    dst3 = dst.reshape(E // T_EDGE, 1, T_EDGE)
    u, wb, didx2 = _edge_call(pl_rel, ksvs, qd, dst3, p)
    idx_cat = didx2.reshape(2 * E)
    z = jnp.zeros((STRIPE, HID), jnp.float32)
    scat = _make_scatter()
    agg2 = scat(u, idx_cat, z)
    den2 = scat(wb, idx_cat, z)
    y = _final_call(agg2, den2, xd, d, p)
    return y.reshape(dst_x.shape)


# speedup vs baseline: 25.5386x; 1.0870x over previous
import functools

import jax
import jax.numpy as jnp
from jax import lax
from jax.experimental import pallas as pl
from jax.experimental.pallas import tpu as pltpu
from jax.experimental.pallas import tpu_sc as plsc

HID = 128
NFREQ = 64
NHEAD = 8
HDIM = HID // NHEAD
N_PL = 10000
N_DST = 64 * 6 * 60
E = 368640
LNEPS = 1e-5
SCALE = HDIM ** -0.5

C_EDGE = 128
T_EDGE = 1024
T_DST = 384
HN = N_DST // 2
NR = 11648
STRIPE = NR // 16


def _ln(x):
    mu = x.mean(-1, keepdims=True)
    var = ((x - mu) ** 2).mean(-1, keepdims=True)
    return (x - mu) * lax.rsqrt(var + LNEPS)


_COSC = (0.9999994434155783, -19.73903432200607, 64.93061147431378,
         -85.29594600637873, 58.91242234401621, -21.28277632550919)
_SINC = (6.283185031925637, -41.34161601075037, 81.60091294958659,
         -76.62654276523628, 41.40338719023908, -12.576281257548619)


def _poly(s, coeffs):
    acc = jnp.full_like(s, coeffs[-1])
    for c in coeffs[-2::-1]:
        acc = acc * s + c
    return acc


def _ln_mxu(x):
    j = jnp.full((HID, HID), 1.0 / HID, jnp.float32)
    m = x @ j
    e2 = (x * x) @ j
    var = e2 - m * m
    return (x - m) * lax.rsqrt(var + LNEPS)


def _src_prep_body(mx_ref, wk_ref, wv_ref, kv_ref):
    xs = _ln(mx_ref[...])
    kv_ref[...] = jnp.concatenate([xs @ wk_ref[...], xs @ wv_ref[...]], axis=1)


def _dst_prep_body(d_ref, wq_ref, bq_ref, q_ref, xd_ref):
    xd = _ln(d_ref[...])
    q_ref[...] = xd @ wq_ref[...] + bq_ref[...]
    xd_ref[...] = xd


def _node_prep(map_x, d, p):
    kv = pl.pallas_call(
        _src_prep_body,
        grid=(10,),
        in_specs=[
            pl.BlockSpec((N_PL // 10, HID), lambda i: (i, 0)),
            pl.BlockSpec((HID, HID), lambda i: (0, 0)),
            pl.BlockSpec((HID, HID), lambda i: (0, 0)),
        ],
        out_specs=pl.BlockSpec((N_PL // 10, 2 * HID), lambda i: (i, 0)),
        out_shape=jax.ShapeDtypeStruct((N_PL, 2 * HID), jnp.float32),
    )(map_x, p['wk'], p['wv'])
    q, xd = pl.pallas_call(
        _dst_prep_body,
        grid=(N_DST // T_DST,),
        in_specs=[
            pl.BlockSpec((T_DST, HID), lambda i: (i, 0)),
            pl.BlockSpec((HID, HID), lambda i: (0, 0)),
            pl.BlockSpec((1, HID), lambda i: (0, 0)),
        ],
        out_specs=[
            pl.BlockSpec((T_DST, HID), lambda i: (i, 0)),
            pl.BlockSpec((T_DST, HID), lambda i: (i, 0)),
        ],
        out_shape=[
            jax.ShapeDtypeStruct((N_DST, HID), jnp.float32),
            jax.ShapeDtypeStruct((N_DST, HID), jnp.float32),
        ],
    )(d, p['wq'], p['bq'].reshape(1, HID))
    return kv, q, xd


def _make_gather():
    mesh = plsc.VectorSubcoreMesh(core_axis_name="c", subcore_axis_name="s")

    @functools.partial(
        pl.kernel,
        mesh=mesh,
        out_type=[
            jax.ShapeDtypeStruct((E, 2 * HID), jnp.float32),
            jax.ShapeDtypeStruct((E, HID), jnp.float32),
        ],
        scratch_types=[
            pltpu.VMEM((C_EDGE,), jnp.int32),
            pltpu.VMEM((C_EDGE,), jnp.int32),
            pltpu.VMEM((C_EDGE, 2 * HID), jnp.float32),
            pltpu.VMEM((C_EDGE, HID), jnp.float32),
            pltpu.SemaphoreType.DMA,
            pltpu.SemaphoreType.DMA,
        ],
    )
    def gath(kv_hbm, q_hbm, src_hbm, dst_hbm, ksvs_hbm, qd_hbm,
             sidx, didx, kvbuf, qbuf, sem1, sem2):
        wid = lax.axis_index("s") * 2 + lax.axis_index("c")
        per = E // 32

        def body(i, carry):
            base = wid * per + i * C_EDGE
            pltpu.sync_copy(src_hbm.at[pl.ds(base, C_EDGE)], sidx)
            pltpu.sync_copy(dst_hbm.at[pl.ds(base, C_EDGE)], didx)
            cp1 = pltpu.async_copy(kv_hbm.at[sidx], kvbuf, sem1)
            cp2 = pltpu.async_copy(q_hbm.at[didx], qbuf, sem2)
            cp1.wait()
            cp2.wait()
            pltpu.sync_copy(kvbuf, ksvs_hbm.at[pl.ds(base, C_EDGE)])
            pltpu.sync_copy(qbuf, qd_hbm.at[pl.ds(base, C_EDGE)])
            return carry

        lax.fori_loop(0, per // C_EDGE, body, 0)

    return gath


def _edge_body(rel_ref, ksvs_ref, qd_ref, dst_ref, freqs_ref, w1_ref, b1_ref,
               w2_ref, wo_ref, bo_ref, wkr_ref, wvr_ref, u_ref, wb_ref,
               didx_ref):
    rel = rel_ref[...]
    acc = jnp.zeros((rel.shape[0], HID), jnp.float32)
    freqs = freqs_ref[...]
    for i in range(3):
        xi = rel[:, i:i + 1]
        z = xi * freqs[i][None, :]
        r = z - jnp.round(z)
        sq = r * r
        cosf = _poly(sq, _COSC)
        sinf = r * _poly(sq, _SINC)
        w1 = w1_ref[...][i]
        hi = (cosf @ w1[:NFREQ]
              + sinf @ w1[NFREQ:2 * NFREQ]
              + xi * w1[2 * NFREQ][None, :]
              + b1_ref[...][i][None, :])
        hi = jnp.maximum(_ln_mxu(hi), 0.0)
        acc = acc + hi @ w2_ref[...][i]
    relh = jnp.maximum(_ln_mxu(acc), 0.0) @ wo_ref[...] + bo_ref[...]
    r = _ln_mxu(relh)
    kr = r @ wkr_ref[...]
    vr = r @ wvr_ref[...]
    ksvs = ksvs_ref[...]
    k_e = ksvs[:, :HID] + kr
    v_e = ksvs[:, HID:] + vr
    prod = qd_ref[...] * k_e
    hsel = (lax.broadcasted_iota(jnp.int32, (HID, NHEAD), 0) // HDIM
            == lax.broadcasted_iota(jnp.int32, (HID, NHEAD), 1))
    bd = hsel.astype(jnp.float32)
    sim = (prod @ bd) * SCALE
    w128 = jnp.exp(sim) @ bd.T
    u_ref[...] = w128 * v_e
    wb_ref[...] = w128
    dstv = dst_ref[0]
    dump = HN + (lax.broadcasted_iota(jnp.int32, dstv.shape, 1) % 64)
    d0 = jnp.where(dstv < HN, dstv, dump)
    d1 = jnp.where(dstv >= HN, dstv - HN, dump)
    didx_ref[...] = jnp.concatenate([d0, d1], axis=0)


def _edge_call(pl_rel, ksvs, qd, dst3, p):
    full = lambda shape: pl.BlockSpec(shape, lambda i: tuple(0 for _ in shape))
    return pl.pallas_call(
        _edge_body,
        grid=(E // T_EDGE,),
        in_specs=[
            pl.BlockSpec((T_EDGE, 3), lambda i: (i, 0)),
            pl.BlockSpec((T_EDGE, 2 * HID), lambda i: (i, 0)),
            pl.BlockSpec((T_EDGE, HID), lambda i: (i, 0)),
            pl.BlockSpec((1, 1, T_EDGE), lambda i: (i, 0, 0)),
            full((3, NFREQ)),
            full((3, 2 * NFREQ + 1, HID)),
            full((3, HID)),
            full((3, HID, HID)),
            full((HID, HID)),
            full((1, HID)),
            full((HID, HID)),
            full((HID, HID)),
        ],
        out_specs=[
            pl.BlockSpec((T_EDGE, HID), lambda i: (i, 0)),
            pl.BlockSpec((T_EDGE, HID), lambda i: (i, 0)),
            pl.BlockSpec((2, T_EDGE), lambda i: (0, i)),
        ],
        out_shape=[
            jax.ShapeDtypeStruct((E, HID), jnp.float32),
            jax.ShapeDtypeStruct((E, HID), jnp.float32),
            jax.ShapeDtypeStruct((2, E), jnp.int32),
        ],
    )(pl_rel, ksvs, qd, dst3, p['fe_freqs'], p['fe_w1'], p['fe_b1'],
      p['fe_w2'], p['fe_wo'], p['fe_bo'].reshape(1, HID), p['wkr'], p['wvr'])


def _make_scatter():
    mesh = plsc.VectorSubcoreMesh(core_axis_name="c", subcore_axis_name="s")

    @functools.partial(
        pl.kernel,
        mesh=mesh,
        out_type=jax.ShapeDtypeStruct((2, NR, HID), jnp.float32),
        scratch_types=[
            pltpu.VMEM((C_EDGE,), jnp.int32),
            pltpu.VMEM((C_EDGE, HID), jnp.float32),
            pltpu.VMEM_SHARED((NR, HID), jnp.float32),
        ],
    )
    def scat(vals_hbm, idx_hbm, z_hbm, out_hbm, didx, vbuf, acc):
        c = lax.axis_index("c")
        s = lax.axis_index("s")
        pltpu.sync_copy(z_hbm, acc.at[pl.ds(s * STRIPE, STRIPE)])
        plsc.subcore_barrier()
        per = E // 16

        def body(i, carry):
            base = s * per + i * C_EDGE
            pltpu.sync_copy(idx_hbm.at[pl.ds(c * E + base, C_EDGE)], didx)
            pltpu.sync_copy(vals_hbm.at[pl.ds(base, C_EDGE)], vbuf)
            pltpu.sync_copy(vbuf, acc.at[didx], add=True)
            return carry

        lax.fori_loop(0, per // C_EDGE, body, 0)
        plsc.subcore_barrier()
        pltpu.sync_copy(acc.at[pl.ds(s * STRIPE, STRIPE)],
                        out_hbm.at[c, pl.ds(s * STRIPE, STRIPE)])

    return scat


def _final_body(agg_ref, den_ref, xd_ref, draw_ref,
                wg_ref, bg_ref, ws_ref, bs_ref, wo_ref, bo_ref, w1_ref, b1_ref,
                w2_ref, b2_ref, out_ref):
    agg_u = agg_ref[0]
    den128 = den_ref[0]
    agg = agg_u / (den128 + 1e-9)
    xd = xd_ref[...]
    wg = wg_ref[...]
    g = jax.nn.sigmoid(agg @ wg[:HID] + xd @ wg[HID:] + bg_ref[...])
    sv = xd @ ws_ref[...] + bs_ref[...]
    outv = agg + g * (sv - agg)
    x = draw_ref[...] + outv @ wo_ref[...] + bo_ref[...]
    h = _ln(x)
    out_ref[...] = (x + jnp.maximum(h @ w1_ref[...] + b1_ref[...], 0.0)
                    @ w2_ref[...] + b2_ref[...])


def _final_call(agg2, den2, xd, d, p):
    full = lambda shape: pl.BlockSpec(shape, lambda i: tuple(0 for _ in shape))
    row = lambda w: pl.BlockSpec((T_DST, w), lambda i: (i, 0))
    nhalf = HN // T_DST
    acc_spec = pl.BlockSpec((1, T_DST, HID),
                            lambda i: (i // nhalf, i % nhalf, 0))
    return pl.pallas_call(
        _final_body,
        grid=(N_DST // T_DST,),
        in_specs=[
            acc_spec, acc_spec,
            row(HID), row(HID),
            full((2 * HID, HID)), full((1, HID)),
            full((HID, HID)), full((1, HID)),
            full((HID, HID)), full((1, HID)),
            full((HID, 4 * HID)), full((1, 4 * HID)),
            full((4 * HID, HID)), full((1, HID)),
        ],
        out_specs=pl.BlockSpec((T_DST, HID), lambda i: (i, 0)),
        out_shape=jax.ShapeDtypeStruct((N_DST, HID), jnp.float32),
    )(agg2, den2, xd, d,
      p['wg'], p['bg'].reshape(1, HID),
      p['ws'], p['bs'].reshape(1, HID),
      p['wo'], p['bo'].reshape(1, HID),
      p['w1'], p['b1'].reshape(1, 4 * HID),
      p['w2'], p['b2'].reshape(1, HID))


def kernel(map_x, edges, pl_rel, dst_x, params):
    p = params
    src = edges[0]
    dst = edges[1]
    d = dst_x.reshape(N_DST, HID)

    kv, q, xd = _node_prep(map_x, d, p)
    ksvs, qd = _make_gather()(kv, q, src, dst)
    dst3 = dst.reshape(E // T_EDGE, 1, T_EDGE)
    u, wb, didx2 = _edge_call(pl_rel, ksvs, qd, dst3, p)
    idx_cat = didx2.reshape(2 * E)
    z = jnp.zeros((STRIPE, HID), jnp.float32)
    scat = _make_scatter()
    agg2 = scat(u, idx_cat, z)
    den2 = scat(wb, idx_cat, z)
    y = _final_call(agg2, den2, xd, d, p)
    return y.reshape(dst_x.shape)


# speedup vs baseline: 30.7872x; 1.2055x over previous
import functools

import jax
import jax.numpy as jnp
from jax import lax
from jax.experimental import pallas as pl
from jax.experimental.pallas import tpu as pltpu
from jax.experimental.pallas import tpu_sc as plsc

HID = 128
NFREQ = 64
NHEAD = 8
HDIM = HID // NHEAD
N_PL = 10000
N_DST = 64 * 6 * 60
E = 368640
LNEPS = 1e-5
SCALE = HDIM ** -0.5

C_EDGE = 128
T_EDGE = 1024
T_DST = 384
HN = N_DST // 2
NR = 11648
STRIPE = NR // 16


def _ln(x):
    mu = x.mean(-1, keepdims=True)
    var = ((x - mu) ** 2).mean(-1, keepdims=True)
    return (x - mu) * lax.rsqrt(var + LNEPS)


_COSC = (0.9999994434155783, -19.73903432200607, 64.93061147431378,
         -85.29594600637873, 58.91242234401621, -21.28277632550919)
_SINC = (6.283185031925637, -41.34161601075037, 81.60091294958659,
         -76.62654276523628, 41.40338719023908, -12.576281257548619)


def _poly(s, coeffs):
    acc = jnp.full_like(s, coeffs[-1])
    for c in coeffs[-2::-1]:
        acc = acc * s + c
    return acc


def _ln_mxu(x):
    j = jnp.full((HID, HID), 1.0 / HID, jnp.float32)
    m = x @ j
    e2 = (x * x) @ j
    var = e2 - m * m
    return (x - m) * lax.rsqrt(var + LNEPS)


def _src_prep_body(mx_ref, wk_ref, wv_ref, kv_ref):
    xs = _ln(mx_ref[...])
    kv_ref[...] = jnp.concatenate([xs @ wk_ref[...], xs @ wv_ref[...]], axis=1)


def _dst_prep_body(d_ref, wq_ref, bq_ref, q_ref, xd_ref):
    xd = _ln(d_ref[...])
    q_ref[...] = xd @ wq_ref[...] + bq_ref[...]
    xd_ref[...] = xd


def _node_prep(map_x, d, p):
    kv = pl.pallas_call(
        _src_prep_body,
        grid=(10,),
        in_specs=[
            pl.BlockSpec((N_PL // 10, HID), lambda i: (i, 0)),
            pl.BlockSpec((HID, HID), lambda i: (0, 0)),
            pl.BlockSpec((HID, HID), lambda i: (0, 0)),
        ],
        out_specs=pl.BlockSpec((N_PL // 10, 2 * HID), lambda i: (i, 0)),
        out_shape=jax.ShapeDtypeStruct((N_PL, 2 * HID), jnp.float32),
    )(map_x, p['wk'], p['wv'])
    q, xd = pl.pallas_call(
        _dst_prep_body,
        grid=(N_DST // T_DST,),
        in_specs=[
            pl.BlockSpec((T_DST, HID), lambda i: (i, 0)),
            pl.BlockSpec((HID, HID), lambda i: (0, 0)),
            pl.BlockSpec((1, HID), lambda i: (0, 0)),
        ],
        out_specs=[
            pl.BlockSpec((T_DST, HID), lambda i: (i, 0)),
            pl.BlockSpec((T_DST, HID), lambda i: (i, 0)),
        ],
        out_shape=[
            jax.ShapeDtypeStruct((N_DST, HID), jnp.float32),
            jax.ShapeDtypeStruct((N_DST, HID), jnp.float32),
        ],
    )(d, p['wq'], p['bq'].reshape(1, HID))
    return kv, q, xd


def _make_gather():
    mesh = plsc.VectorSubcoreMesh(core_axis_name="c", subcore_axis_name="s")

    @functools.partial(
        pl.kernel,
        mesh=mesh,
        out_type=[
            jax.ShapeDtypeStruct((E, 2 * HID), jnp.float32),
            jax.ShapeDtypeStruct((E, HID), jnp.float32),
        ],
        scratch_types=[
            pltpu.VMEM((C_EDGE,), jnp.int32),
            pltpu.VMEM((C_EDGE,), jnp.int32),
            pltpu.VMEM((C_EDGE, 2 * HID), jnp.float32),
            pltpu.VMEM((C_EDGE, HID), jnp.float32),
            pltpu.SemaphoreType.DMA,
            pltpu.SemaphoreType.DMA,
        ],
    )
    def gath(kv_hbm, q_hbm, src_hbm, dst_hbm, ksvs_hbm, qd_hbm,
             sidx, didx, kvbuf, qbuf, sem1, sem2):
        wid = lax.axis_index("s") * 2 + lax.axis_index("c")
        per = E // 32

        def body(i, carry):
            base = wid * per + i * C_EDGE
            pltpu.sync_copy(src_hbm.at[pl.ds(base, C_EDGE)], sidx)
            pltpu.sync_copy(dst_hbm.at[pl.ds(base, C_EDGE)], didx)
            cp1 = pltpu.async_copy(kv_hbm.at[sidx], kvbuf, sem1)
            cp2 = pltpu.async_copy(q_hbm.at[didx], qbuf, sem2)
            cp1.wait()
            cp2.wait()
            pltpu.sync_copy(kvbuf, ksvs_hbm.at[pl.ds(base, C_EDGE)])
            pltpu.sync_copy(qbuf, qd_hbm.at[pl.ds(base, C_EDGE)])
            return carry

        lax.fori_loop(0, per // C_EDGE, body, 0)

    return gath


def _edge_body(rel_ref, ksvs_ref, qd_ref, dst_ref, freqs_ref, w1_ref, b1_ref,
               w2_ref, wo_ref, bo_ref, wkr_ref, wvr_ref, u_ref, wb_ref,
               didx_ref):
    rel = rel_ref[...]
    acc = jnp.zeros((rel.shape[0], HID), jnp.float32)
    freqs = freqs_ref[...]
    for i in range(3):
        xi = rel[:, i:i + 1]
        z = xi * freqs[i][None, :]
        r = z - jnp.round(z)
        sq = r * r
        cosf = _poly(sq, _COSC)
        sinf = r * _poly(sq, _SINC)
        w1 = w1_ref[...][i]
        hi = (cosf @ w1[:NFREQ]
              + sinf @ w1[NFREQ:2 * NFREQ]
              + xi * w1[2 * NFREQ][None, :]
              + b1_ref[...][i][None, :])
        hi = jnp.maximum(_ln_mxu(hi), 0.0)
        acc = acc + hi @ w2_ref[...][i]
    relh = jnp.maximum(_ln_mxu(acc), 0.0) @ wo_ref[...] + bo_ref[...]
    r = _ln_mxu(relh)
    kr = r @ wkr_ref[...]
    vr = r @ wvr_ref[...]
    ksvs = ksvs_ref[...]
    k_e = ksvs[:, :HID] + kr
    v_e = ksvs[:, HID:] + vr
    prod = qd_ref[...] * k_e
    hsel = (lax.broadcasted_iota(jnp.int32, (HID, NHEAD), 0) // HDIM
            == lax.broadcasted_iota(jnp.int32, (HID, NHEAD), 1))
    bd = hsel.astype(jnp.float32)
    sim = (prod @ bd) * SCALE
    w128 = jnp.exp(sim) @ bd.T
    u_ref[...] = w128 * v_e
    wb_ref[...] = w128
    dstv = dst_ref[0]
    dump = HN + (lax.broadcasted_iota(jnp.int32, dstv.shape, 1) % 64)
    d0 = jnp.where(dstv < HN, dstv, dump)
    d1 = jnp.where(dstv >= HN, dstv - HN, dump)
    didx_ref[...] = jnp.concatenate([d0, d1], axis=0)


def _edge_call(pl_rel, ksvs, qd, dst3, p):
    full = lambda shape: pl.BlockSpec(shape, lambda i: tuple(0 for _ in shape))
    return pl.pallas_call(
        _edge_body,
        grid=(E // T_EDGE,),
        in_specs=[
            pl.BlockSpec((T_EDGE, 3), lambda i: (i, 0)),
            pl.BlockSpec((T_EDGE, 2 * HID), lambda i: (i, 0)),
            pl.BlockSpec((T_EDGE, HID), lambda i: (i, 0)),
            pl.BlockSpec((1, 1, T_EDGE), lambda i: (i, 0, 0)),
            full((3, NFREQ)),
            full((3, 2 * NFREQ + 1, HID)),
            full((3, HID)),
            full((3, HID, HID)),
            full((HID, HID)),
            full((1, HID)),
            full((HID, HID)),
            full((HID, HID)),
        ],
        out_specs=[
            pl.BlockSpec((T_EDGE, HID), lambda i: (i, 0)),
            pl.BlockSpec((T_EDGE, HID), lambda i: (i, 0)),
            pl.BlockSpec((2, T_EDGE), lambda i: (0, i)),
        ],
        out_shape=[
            jax.ShapeDtypeStruct((E, HID), jnp.float32),
            jax.ShapeDtypeStruct((E, HID), jnp.float32),
            jax.ShapeDtypeStruct((2, E), jnp.int32),
        ],
    )(pl_rel, ksvs, qd, dst3, p['fe_freqs'], p['fe_w1'], p['fe_b1'],
      p['fe_w2'], p['fe_wo'], p['fe_bo'].reshape(1, HID), p['wkr'], p['wvr'])


def _make_scatter():
    mesh = plsc.VectorSubcoreMesh(core_axis_name="c", subcore_axis_name="s")

    @functools.partial(
        pl.kernel,
        mesh=mesh,
        out_type=jax.ShapeDtypeStruct((2, NR, HID), jnp.float32),
        scratch_types=[
            pltpu.VMEM((C_EDGE,), jnp.int32),
            pltpu.VMEM((C_EDGE,), jnp.int32),
            pltpu.VMEM((C_EDGE, HID), jnp.float32),
            pltpu.VMEM((C_EDGE, HID), jnp.float32),
            pltpu.SemaphoreType.DMA,
            pltpu.SemaphoreType.DMA,
            pltpu.VMEM_SHARED((NR, HID), jnp.float32),
        ],
    )
    def scat(vals_hbm, idx_hbm, z_hbm, out_hbm, didx0, didx1, vbuf0, vbuf1,
             sem0, sem1, acc):
        c = lax.axis_index("c")
        s = lax.axis_index("s")
        pltpu.sync_copy(z_hbm, acc.at[pl.ds(s * STRIPE, STRIPE)])
        plsc.subcore_barrier()
        per = E // 16
        nch = per // C_EDGE
        bufs = ((didx0, vbuf0, sem0), (didx1, vbuf1, sem1))

        def load(i, didx, vbuf, sem):
            base = s * per + i * C_EDGE
            pltpu.async_copy(idx_hbm.at[pl.ds(c * E + base, C_EDGE)],
                             didx, sem)
            pltpu.async_copy(vals_hbm.at[pl.ds(base, C_EDGE)], vbuf, sem)

        def drain(didx, vbuf, sem):
            pltpu.make_async_copy(idx_hbm.at[pl.ds(0, C_EDGE)],
                                  didx, sem).wait()
            pltpu.make_async_copy(vals_hbm.at[pl.ds(0, C_EDGE)],
                                  vbuf, sem).wait()

        load(0, *bufs[0])

        def body(g, carry):
            for b in range(2):
                i = 2 * g + b
                didx, vbuf, sem = bufs[b]
                ndidx, nvbuf, nsem = bufs[1 - b]

                @pl.when(i + 1 < nch)
                def _():
                    load(i + 1, ndidx, nvbuf, nsem)

                drain(didx, vbuf, sem)
                pltpu.sync_copy(vbuf, acc.at[didx], add=True)
            return carry

        lax.fori_loop(0, nch // 2, body, 0)
        plsc.subcore_barrier()
        pltpu.sync_copy(acc.at[pl.ds(s * STRIPE, STRIPE)],
                        out_hbm.at[c, pl.ds(s * STRIPE, STRIPE)])

    return scat


def _final_body(agg_ref, den_ref, xd_ref, draw_ref,
                wg_ref, bg_ref, ws_ref, bs_ref, wo_ref, bo_ref, w1_ref, b1_ref,
                w2_ref, b2_ref, out_ref):
    agg_u = agg_ref[0]
    den128 = den_ref[0]
    agg = agg_u / (den128 + 1e-9)
    xd = xd_ref[...]
    wg = wg_ref[...]
    g = jax.nn.sigmoid(agg @ wg[:HID] + xd @ wg[HID:] + bg_ref[...])
    sv = xd @ ws_ref[...] + bs_ref[...]
    outv = agg + g * (sv - agg)
    x = draw_ref[...] + outv @ wo_ref[...] + bo_ref[...]
    h = _ln(x)
    out_ref[...] = (x + jnp.maximum(h @ w1_ref[...] + b1_ref[...], 0.0)
                    @ w2_ref[...] + b2_ref[...])


def _final_call(agg2, den2, xd, d, p):
    full = lambda shape: pl.BlockSpec(shape, lambda i: tuple(0 for _ in shape))
    row = lambda w: pl.BlockSpec((T_DST, w), lambda i: (i, 0))
    nhalf = HN // T_DST
    acc_spec = pl.BlockSpec((1, T_DST, HID),
                            lambda i: (i // nhalf, i % nhalf, 0))
    return pl.pallas_call(
        _final_body,
        grid=(N_DST // T_DST,),
        in_specs=[
            acc_spec, acc_spec,
            row(HID), row(HID),
            full((2 * HID, HID)), full((1, HID)),
            full((HID, HID)), full((1, HID)),
            full((HID, HID)), full((1, HID)),
            full((HID, 4 * HID)), full((1, 4 * HID)),
            full((4 * HID, HID)), full((1, HID)),
        ],
        out_specs=pl.BlockSpec((T_DST, HID), lambda i: (i, 0)),
        out_shape=jax.ShapeDtypeStruct((N_DST, HID), jnp.float32),
    )(agg2, den2, xd, d,
      p['wg'], p['bg'].reshape(1, HID),
      p['ws'], p['bs'].reshape(1, HID),
      p['wo'], p['bo'].reshape(1, HID),
      p['w1'], p['b1'].reshape(1, 4 * HID),
      p['w2'], p['b2'].reshape(1, HID))


def kernel(map_x, edges, pl_rel, dst_x, params):
    p = params
    src = edges[0]
    dst = edges[1]
    d = dst_x.reshape(N_DST, HID)

    kv, q, xd = _node_prep(map_x, d, p)
    ksvs, qd = _make_gather()(kv, q, src, dst)
    dst3 = dst.reshape(E // T_EDGE, 1, T_EDGE)
    u, wb, didx2 = _edge_call(pl_rel, ksvs, qd, dst3, p)
    idx_cat = didx2.reshape(2 * E)
    z = jnp.zeros((STRIPE, HID), jnp.float32)
    scat = _make_scatter()
    agg2 = scat(u, idx_cat, z)
    den2 = scat(wb, idx_cat, z)
    y = _final_call(agg2, den2, xd, d, p)
    return y.reshape(dst_x.shape)


# speedup vs baseline: 32.1564x; 1.0445x over previous
import functools

import jax
import jax.numpy as jnp
from jax import lax
from jax.experimental import pallas as pl
from jax.experimental.pallas import tpu as pltpu
from jax.experimental.pallas import tpu_sc as plsc

HID = 128
NFREQ = 64
NHEAD = 8
HDIM = HID // NHEAD
N_PL = 10000
N_DST = 64 * 6 * 60
E = 368640
LNEPS = 1e-5
SCALE = HDIM ** -0.5

C_EDGE = 128
T_EDGE = 1024
T_DST = 384
HN = N_DST // 2
NR = 11648
STRIPE = NR // 16


def _ln(x):
    mu = x.mean(-1, keepdims=True)
    var = ((x - mu) ** 2).mean(-1, keepdims=True)
    return (x - mu) * lax.rsqrt(var + LNEPS)


_COSC = (0.9999994434155783, -19.73903432200607, 64.93061147431378,
         -85.29594600637873, 58.91242234401621, -21.28277632550919)
_SINC = (6.283185031925637, -41.34161601075037, 81.60091294958659,
         -76.62654276523628, 41.40338719023908, -12.576281257548619)


def _poly(s, coeffs):
    acc = jnp.full_like(s, coeffs[-1])
    for c in coeffs[-2::-1]:
        acc = acc * s + c
    return acc


def _ln_mxu(x):
    j = jnp.full((HID, HID), 1.0 / HID, jnp.float32)
    m = x @ j
    e2 = (x * x) @ j
    var = e2 - m * m
    return (x - m) * lax.rsqrt(var + LNEPS)


def _src_prep_body(mx_ref, wk_ref, wv_ref, kv_ref):
    xs = _ln(mx_ref[...])
    kv_ref[...] = jnp.concatenate([xs @ wk_ref[...], xs @ wv_ref[...]], axis=1)


def _dst_prep_body(d_ref, wq_ref, bq_ref, q_ref, xd_ref):
    xd = _ln(d_ref[...])
    q_ref[...] = xd @ wq_ref[...] + bq_ref[...]
    xd_ref[...] = xd


def _node_prep(map_x, d, p):
    kv = pl.pallas_call(
        _src_prep_body,
        grid=(10,),
        in_specs=[
            pl.BlockSpec((N_PL // 10, HID), lambda i: (i, 0)),
            pl.BlockSpec((HID, HID), lambda i: (0, 0)),
            pl.BlockSpec((HID, HID), lambda i: (0, 0)),
        ],
        out_specs=pl.BlockSpec((N_PL // 10, 2 * HID), lambda i: (i, 0)),
        out_shape=jax.ShapeDtypeStruct((N_PL, 2 * HID), jnp.float32),
    )(map_x, p['wk'], p['wv'])
    q, xd = pl.pallas_call(
        _dst_prep_body,
        grid=(N_DST // T_DST,),
        in_specs=[
            pl.BlockSpec((T_DST, HID), lambda i: (i, 0)),
            pl.BlockSpec((HID, HID), lambda i: (0, 0)),
            pl.BlockSpec((1, HID), lambda i: (0, 0)),
        ],
        out_specs=[
            pl.BlockSpec((T_DST, HID), lambda i: (i, 0)),
            pl.BlockSpec((T_DST, HID), lambda i: (i, 0)),
        ],
        out_shape=[
            jax.ShapeDtypeStruct((N_DST, HID), jnp.float32),
            jax.ShapeDtypeStruct((N_DST, HID), jnp.float32),
        ],
    )(d, p['wq'], p['bq'].reshape(1, HID))
    return kv, q, xd


def _make_gather():
    mesh = plsc.VectorSubcoreMesh(core_axis_name="c", subcore_axis_name="s")

    @functools.partial(
        pl.kernel,
        mesh=mesh,
        out_type=[
            jax.ShapeDtypeStruct((E, 2 * HID), jnp.float32),
            jax.ShapeDtypeStruct((E, HID), jnp.float32),
        ],
        scratch_types=[
            pltpu.VMEM((2, C_EDGE), jnp.int32),
            pltpu.VMEM((2, C_EDGE), jnp.int32),
            pltpu.VMEM((C_EDGE, 2 * HID), jnp.float32),
            pltpu.VMEM((C_EDGE, 2 * HID), jnp.float32),
            pltpu.VMEM((C_EDGE, HID), jnp.float32),
            pltpu.VMEM((C_EDGE, HID), jnp.float32),
            pltpu.SemaphoreType.DMA,
            pltpu.SemaphoreType.DMA,
            pltpu.SemaphoreType.DMA,
            pltpu.SemaphoreType.DMA,
            pltpu.SemaphoreType.DMA,
            pltpu.SemaphoreType.DMA,
        ],
    )
    def gath(kv_hbm, q_hbm, src_hbm, dst_hbm, ksvs_hbm, qd_hbm,
             sidx2, didx2, kvbuf0, kvbuf1, qbuf0, qbuf1,
             semi0, semi1, semg0, semg1, semw0, semw1):
        wid = lax.axis_index("s") * 2 + lax.axis_index("c")
        per = E // 32
        nch = per // C_EDGE
        bufs = ((kvbuf0, qbuf0, semi0, semg0, semw0),
                (kvbuf1, qbuf1, semi1, semg1, semw1))

        def load_idx(i, b, semi):
            base = wid * per + i * C_EDGE
            pltpu.async_copy(src_hbm.at[pl.ds(base, C_EDGE)],
                             sidx2.at[b], semi)
            pltpu.async_copy(dst_hbm.at[pl.ds(base, C_EDGE)],
                             didx2.at[b], semi)

        def drain_idx(b, semi):
            pltpu.make_async_copy(src_hbm.at[pl.ds(0, C_EDGE)],
                                  sidx2.at[b], semi).wait()
            pltpu.make_async_copy(dst_hbm.at[pl.ds(0, C_EDGE)],
                                  didx2.at[b], semi).wait()

        load_idx(0, 0, semi0)
        load_idx(1, 1, semi1)

        def body(g, carry):
            for b in range(2):
                i = 2 * g + b
                kvbuf, qbuf, semi, semg, semw = bufs[b]

                @pl.when(i >= 2)
                def _():
                    pltpu.make_async_copy(kvbuf, ksvs_hbm.at[pl.ds(0, C_EDGE)],
                                          semw).wait()
                    pltpu.make_async_copy(qbuf, qd_hbm.at[pl.ds(0, C_EDGE)],
                                          semw).wait()

                drain_idx(b, semi)
                pltpu.async_copy(kv_hbm.at[sidx2.at[b]], kvbuf, semg)
                pltpu.async_copy(q_hbm.at[didx2.at[b]], qbuf, semg)
                pltpu.make_async_copy(kv_hbm.at[sidx2.at[b]], kvbuf,
                                      semg).wait()
                pltpu.make_async_copy(q_hbm.at[didx2.at[b]], qbuf,
                                      semg).wait()

                @pl.when(i + 2 < nch)
                def _():
                    load_idx(i + 2, b, semi)

                base = wid * per + i * C_EDGE
                pltpu.async_copy(kvbuf, ksvs_hbm.at[pl.ds(base, C_EDGE)], semw)
                pltpu.async_copy(qbuf, qd_hbm.at[pl.ds(base, C_EDGE)], semw)
            return carry

        lax.fori_loop(0, nch // 2, body, 0)
        for b in range(2):
            kvbuf, qbuf, semi, semg, semw = bufs[b]
            pltpu.make_async_copy(kvbuf, ksvs_hbm.at[pl.ds(0, C_EDGE)],
                                  semw).wait()
            pltpu.make_async_copy(qbuf, qd_hbm.at[pl.ds(0, C_EDGE)],
                                  semw).wait()

    return gath


def _edge_body(rel_ref, ksvs_ref, qd_ref, dst_ref, freqs_ref, w1_ref, b1_ref,
               w2_ref, wo_ref, bo_ref, wkr_ref, wvr_ref, u_ref, wb_ref,
               didx_ref):
    rel = rel_ref[...]
    acc = jnp.zeros((rel.shape[0], HID), jnp.float32)
    freqs = freqs_ref[...]
    for i in range(3):
        xi = rel[:, i:i + 1]
        z = xi * freqs[i][None, :]
        r = z - jnp.round(z)
        sq = r * r
        cosf = _poly(sq, _COSC)
        sinf = r * _poly(sq, _SINC)
        w1 = w1_ref[...][i]
        hi = (cosf @ w1[:NFREQ]
              + sinf @ w1[NFREQ:2 * NFREQ]
              + xi * w1[2 * NFREQ][None, :]
              + b1_ref[...][i][None, :])
        hi = jnp.maximum(_ln_mxu(hi), 0.0)
        acc = acc + hi @ w2_ref[...][i]
    relh = jnp.maximum(_ln_mxu(acc), 0.0) @ wo_ref[...] + bo_ref[...]
    r = _ln_mxu(relh)
    kr = r @ wkr_ref[...]
    vr = r @ wvr_ref[...]
    ksvs = ksvs_ref[...]
    k_e = ksvs[:, :HID] + kr
    v_e = ksvs[:, HID:] + vr
    prod = qd_ref[...] * k_e
    hsel = (lax.broadcasted_iota(jnp.int32, (HID, NHEAD), 0) // HDIM
            == lax.broadcasted_iota(jnp.int32, (HID, NHEAD), 1))
    bd = hsel.astype(jnp.float32)
    sim = (prod @ bd) * SCALE
    w128 = jnp.exp(sim) @ bd.T
    u_ref[...] = w128 * v_e
    wb_ref[...] = w128
    dstv = dst_ref[0]
    dump = HN + (lax.broadcasted_iota(jnp.int32, dstv.shape, 1) % 64)
    d0 = jnp.where(dstv < HN, dstv, dump)
    d1 = jnp.where(dstv >= HN, dstv - HN, dump)
    didx_ref[...] = jnp.concatenate([d0, d1], axis=0)


def _edge_call(pl_rel, ksvs, qd, dst3, p):
    full = lambda shape: pl.BlockSpec(shape, lambda i: tuple(0 for _ in shape))
    return pl.pallas_call(
        _edge_body,
        grid=(E // T_EDGE,),
        in_specs=[
            pl.BlockSpec((T_EDGE, 3), lambda i: (i, 0)),
            pl.BlockSpec((T_EDGE, 2 * HID), lambda i: (i, 0)),
            pl.BlockSpec((T_EDGE, HID), lambda i: (i, 0)),
            pl.BlockSpec((1, 1, T_EDGE), lambda i: (i, 0, 0)),
            full((3, NFREQ)),
            full((3, 2 * NFREQ + 1, HID)),
            full((3, HID)),
            full((3, HID, HID)),
            full((HID, HID)),
            full((1, HID)),
            full((HID, HID)),
            full((HID, HID)),
        ],
        out_specs=[
            pl.BlockSpec((T_EDGE, HID), lambda i: (i, 0)),
            pl.BlockSpec((T_EDGE, HID), lambda i: (i, 0)),
            pl.BlockSpec((2, T_EDGE), lambda i: (0, i)),
        ],
        out_shape=[
            jax.ShapeDtypeStruct((E, HID), jnp.float32),
            jax.ShapeDtypeStruct((E, HID), jnp.float32),
            jax.ShapeDtypeStruct((2, E), jnp.int32),
        ],
    )(pl_rel, ksvs, qd, dst3, p['fe_freqs'], p['fe_w1'], p['fe_b1'],
      p['fe_w2'], p['fe_wo'], p['fe_bo'].reshape(1, HID), p['wkr'], p['wvr'])


def _make_scatter():
    mesh = plsc.VectorSubcoreMesh(core_axis_name="c", subcore_axis_name="s")

    @functools.partial(
        pl.kernel,
        mesh=mesh,
        out_type=jax.ShapeDtypeStruct((2, NR, HID), jnp.float32),
        scratch_types=[
            pltpu.VMEM((C_EDGE,), jnp.int32),
            pltpu.VMEM((C_EDGE,), jnp.int32),
            pltpu.VMEM((C_EDGE, HID), jnp.float32),
            pltpu.VMEM((C_EDGE, HID), jnp.float32),
            pltpu.SemaphoreType.DMA,
            pltpu.SemaphoreType.DMA,
            pltpu.VMEM_SHARED((NR, HID), jnp.float32),
        ],
    )
    def scat(vals_hbm, idx_hbm, z_hbm, out_hbm, didx0, didx1, vbuf0, vbuf1,
             sem0, sem1, acc):
        c = lax.axis_index("c")
        s = lax.axis_index("s")
        pltpu.sync_copy(z_hbm, acc.at[pl.ds(s * STRIPE, STRIPE)])
        plsc.subcore_barrier()
        per = E // 16
        nch = per // C_EDGE
        bufs = ((didx0, vbuf0, sem0), (didx1, vbuf1, sem1))

        def load(i, didx, vbuf, sem):
            base = s * per + i * C_EDGE
            pltpu.async_copy(idx_hbm.at[pl.ds(c * E + base, C_EDGE)],
                             didx, sem)
            pltpu.async_copy(vals_hbm.at[pl.ds(base, C_EDGE)], vbuf, sem)

        def drain(didx, vbuf, sem):
            pltpu.make_async_copy(idx_hbm.at[pl.ds(0, C_EDGE)],
                                  didx, sem).wait()
            pltpu.make_async_copy(vals_hbm.at[pl.ds(0, C_EDGE)],
                                  vbuf, sem).wait()

        load(0, *bufs[0])

        def body(g, carry):
            for b in range(2):
                i = 2 * g + b
                didx, vbuf, sem = bufs[b]
                ndidx, nvbuf, nsem = bufs[1 - b]

                @pl.when(i + 1 < nch)
                def _():
                    load(i + 1, ndidx, nvbuf, nsem)

                drain(didx, vbuf, sem)
                pltpu.sync_copy(vbuf, acc.at[didx], add=True)
            return carry

        lax.fori_loop(0, nch // 2, body, 0)
        plsc.subcore_barrier()
        pltpu.sync_copy(acc.at[pl.ds(s * STRIPE, STRIPE)],
                        out_hbm.at[c, pl.ds(s * STRIPE, STRIPE)])

    return scat


def _final_body(agg_ref, den_ref, xd_ref, draw_ref,
                wg_ref, bg_ref, ws_ref, bs_ref, wo_ref, bo_ref, w1_ref, b1_ref,
                w2_ref, b2_ref, out_ref):
    agg_u = agg_ref[0]
    den128 = den_ref[0]
    agg = agg_u / (den128 + 1e-9)
    xd = xd_ref[...]
    wg = wg_ref[...]
    g = jax.nn.sigmoid(agg @ wg[:HID] + xd @ wg[HID:] + bg_ref[...])
    sv = xd @ ws_ref[...] + bs_ref[...]
    outv = agg + g * (sv - agg)
    x = draw_ref[...] + outv @ wo_ref[...] + bo_ref[...]
    h = _ln(x)
    out_ref[...] = (x + jnp.maximum(h @ w1_ref[...] + b1_ref[...], 0.0)
                    @ w2_ref[...] + b2_ref[...])


def _final_call(agg2, den2, xd, d, p):
    full = lambda shape: pl.BlockSpec(shape, lambda i: tuple(0 for _ in shape))
    row = lambda w: pl.BlockSpec((T_DST, w), lambda i: (i, 0))
    nhalf = HN // T_DST
    acc_spec = pl.BlockSpec((1, T_DST, HID),
                            lambda i: (i // nhalf, i % nhalf, 0))
    return pl.pallas_call(
        _final_body,
        grid=(N_DST // T_DST,),
        in_specs=[
            acc_spec, acc_spec,
            row(HID), row(HID),
            full((2 * HID, HID)), full((1, HID)),
            full((HID, HID)), full((1, HID)),
            full((HID, HID)), full((1, HID)),
            full((HID, 4 * HID)), full((1, 4 * HID)),
            full((4 * HID, HID)), full((1, HID)),
        ],
        out_specs=pl.BlockSpec((T_DST, HID), lambda i: (i, 0)),
        out_shape=jax.ShapeDtypeStruct((N_DST, HID), jnp.float32),
    )(agg2, den2, xd, d,
      p['wg'], p['bg'].reshape(1, HID),
      p['ws'], p['bs'].reshape(1, HID),
      p['wo'], p['bo'].reshape(1, HID),
      p['w1'], p['b1'].reshape(1, 4 * HID),
      p['w2'], p['b2'].reshape(1, HID))


def kernel(map_x, edges, pl_rel, dst_x, params):
    p = params
    src = edges[0]
    dst = edges[1]
    d = dst_x.reshape(N_DST, HID)

    kv, q, xd = _node_prep(map_x, d, p)
    ksvs, qd = _make_gather()(kv, q, src, dst)
    dst3 = dst.reshape(E // T_EDGE, 1, T_EDGE)
    u, wb, didx2 = _edge_call(pl_rel, ksvs, qd, dst3, p)
    idx_cat = didx2.reshape(2 * E)
    z = jnp.zeros((STRIPE, HID), jnp.float32)
    scat = _make_scatter()
    agg2 = scat(u, idx_cat, z)
    den2 = scat(wb, idx_cat, z)
    y = _final_call(agg2, den2, xd, d, p)
    return y.reshape(dst_x.shape)


# speedup vs baseline: 33.9520x; 1.0558x over previous
import functools

import jax
import jax.numpy as jnp
from jax import lax
from jax.experimental import pallas as pl
from jax.experimental.pallas import tpu as pltpu
from jax.experimental.pallas import tpu_sc as plsc

HID = 128
NFREQ = 64
NHEAD = 8
HDIM = HID // NHEAD
N_PL = 10000
N_DST = 64 * 6 * 60
E = 368640
LNEPS = 1e-5
SCALE = HDIM ** -0.5

C_EDGE = 128
T_EDGE = 2048
T_DST = 384
HN = N_DST // 2
NR = 11648
STRIPE = NR // 16


def _ln(x):
    mu = x.mean(-1, keepdims=True)
    var = ((x - mu) ** 2).mean(-1, keepdims=True)
    return (x - mu) * lax.rsqrt(var + LNEPS)


_COSC = (0.9999994434155783, -19.73903432200607, 64.93061147431378,
         -85.29594600637873, 58.91242234401621, -21.28277632550919)
_SINC = (6.283185031925637, -41.34161601075037, 81.60091294958659,
         -76.62654276523628, 41.40338719023908, -12.576281257548619)


def _poly(s, coeffs):
    acc = jnp.full_like(s, coeffs[-1])
    for c in coeffs[-2::-1]:
        acc = acc * s + c
    return acc


def _ln_mxu(x):
    j = jnp.full((HID, HID), 1.0 / HID, jnp.float32)
    m = x @ j
    e2 = (x * x) @ j
    var = e2 - m * m
    return (x - m) * lax.rsqrt(var + LNEPS)


def _src_prep_body(mx_ref, wk_ref, wv_ref, kv_ref):
    xs = _ln(mx_ref[...])
    kv_ref[...] = jnp.concatenate([xs @ wk_ref[...], xs @ wv_ref[...]], axis=1)


def _dst_prep_body(d_ref, wq_ref, bq_ref, q_ref, xd_ref):
    xd = _ln(d_ref[...])
    q_ref[...] = xd @ wq_ref[...] + bq_ref[...]
    xd_ref[...] = xd


def _node_prep(map_x, d, p):
    kv = pl.pallas_call(
        _src_prep_body,
        grid=(10,),
        in_specs=[
            pl.BlockSpec((N_PL // 10, HID), lambda i: (i, 0)),
            pl.BlockSpec((HID, HID), lambda i: (0, 0)),
            pl.BlockSpec((HID, HID), lambda i: (0, 0)),
        ],
        out_specs=pl.BlockSpec((N_PL // 10, 2 * HID), lambda i: (i, 0)),
        out_shape=jax.ShapeDtypeStruct((N_PL, 2 * HID), jnp.float32),
    )(map_x, p['wk'], p['wv'])
    q, xd = pl.pallas_call(
        _dst_prep_body,
        grid=(N_DST // T_DST,),
        in_specs=[
            pl.BlockSpec((T_DST, HID), lambda i: (i, 0)),
            pl.BlockSpec((HID, HID), lambda i: (0, 0)),
            pl.BlockSpec((1, HID), lambda i: (0, 0)),
        ],
        out_specs=[
            pl.BlockSpec((T_DST, HID), lambda i: (i, 0)),
            pl.BlockSpec((T_DST, HID), lambda i: (i, 0)),
        ],
        out_shape=[
            jax.ShapeDtypeStruct((N_DST, HID), jnp.float32),
            jax.ShapeDtypeStruct((N_DST, HID), jnp.float32),
        ],
    )(d, p['wq'], p['bq'].reshape(1, HID))
    return kv, q, xd


def _make_gather():
    mesh = plsc.VectorSubcoreMesh(core_axis_name="c", subcore_axis_name="s")

    @functools.partial(
        pl.kernel,
        mesh=mesh,
        out_type=[
            jax.ShapeDtypeStruct((E, 2 * HID), jnp.float32),
            jax.ShapeDtypeStruct((E, HID), jnp.float32),
        ],
        scratch_types=[
            pltpu.VMEM((2, C_EDGE), jnp.int32),
            pltpu.VMEM((2, C_EDGE), jnp.int32),
            pltpu.VMEM((C_EDGE, 2 * HID), jnp.float32),
            pltpu.VMEM((C_EDGE, 2 * HID), jnp.float32),
            pltpu.VMEM((C_EDGE, HID), jnp.float32),
            pltpu.VMEM((C_EDGE, HID), jnp.float32),
            pltpu.SemaphoreType.DMA,
            pltpu.SemaphoreType.DMA,
            pltpu.SemaphoreType.DMA,
            pltpu.SemaphoreType.DMA,
            pltpu.SemaphoreType.DMA,
            pltpu.SemaphoreType.DMA,
        ],
    )
    def gath(kv_hbm, q_hbm, src_hbm, dst_hbm, ksvs_hbm, qd_hbm,
             sidx2, didx2, kvbuf0, kvbuf1, qbuf0, qbuf1,
             semi0, semi1, semg0, semg1, semw0, semw1):
        wid = lax.axis_index("s") * 2 + lax.axis_index("c")
        per = E // 32
        nch = per // C_EDGE
        bufs = ((kvbuf0, qbuf0, semi0, semg0, semw0),
                (kvbuf1, qbuf1, semi1, semg1, semw1))

        def load_idx(i, b, semi):
            base = wid * per + i * C_EDGE
            pltpu.async_copy(src_hbm.at[pl.ds(base, C_EDGE)],
                             sidx2.at[b], semi)
            pltpu.async_copy(dst_hbm.at[pl.ds(base, C_EDGE)],
                             didx2.at[b], semi)

        def drain_idx(b, semi):
            pltpu.make_async_copy(src_hbm.at[pl.ds(0, C_EDGE)],
                                  sidx2.at[b], semi).wait()
            pltpu.make_async_copy(dst_hbm.at[pl.ds(0, C_EDGE)],
                                  didx2.at[b], semi).wait()

        load_idx(0, 0, semi0)
        load_idx(1, 1, semi1)

        def body(g, carry):
            for b in range(2):
                i = 2 * g + b
                kvbuf, qbuf, semi, semg, semw = bufs[b]

                @pl.when(i >= 2)
                def _():
                    pltpu.make_async_copy(kvbuf, ksvs_hbm.at[pl.ds(0, C_EDGE)],
                                          semw).wait()
                    pltpu.make_async_copy(qbuf, qd_hbm.at[pl.ds(0, C_EDGE)],
                                          semw).wait()

                drain_idx(b, semi)
                pltpu.async_copy(kv_hbm.at[sidx2.at[b]], kvbuf, semg)
                pltpu.async_copy(q_hbm.at[didx2.at[b]], qbuf, semg)
                pltpu.make_async_copy(kv_hbm.at[sidx2.at[b]], kvbuf,
                                      semg).wait()
                pltpu.make_async_copy(q_hbm.at[didx2.at[b]], qbuf,
                                      semg).wait()

                @pl.when(i + 2 < nch)
                def _():
                    load_idx(i + 2, b, semi)

                base = wid * per + i * C_EDGE
                pltpu.async_copy(kvbuf, ksvs_hbm.at[pl.ds(base, C_EDGE)], semw)
                pltpu.async_copy(qbuf, qd_hbm.at[pl.ds(base, C_EDGE)], semw)
            return carry

        lax.fori_loop(0, nch // 2, body, 0)
        for b in range(2):
            kvbuf, qbuf, semi, semg, semw = bufs[b]
            pltpu.make_async_copy(kvbuf, ksvs_hbm.at[pl.ds(0, C_EDGE)],
                                  semw).wait()
            pltpu.make_async_copy(qbuf, qd_hbm.at[pl.ds(0, C_EDGE)],
                                  semw).wait()

    return gath


def _edge_body(rel_ref, ksvs_ref, qd_ref, dst_ref, freqs_ref, w1_ref, b1_ref,
               w2_ref, wo_ref, bo_ref, wkr_ref, wvr_ref, u_ref, wb_ref,
               didx_ref):
    rel = rel_ref[...]
    acc = jnp.zeros((rel.shape[0], HID), jnp.float32)
    freqs = freqs_ref[...]
    for i in range(3):
        xi = rel[:, i:i + 1]
        z = xi * freqs[i][None, :]
        r = z - jnp.round(z)
        sq = r * r
        cosf = _poly(sq, _COSC)
        sinf = r * _poly(sq, _SINC)
        w1 = w1_ref[...][i]
        hi = (cosf @ w1[:NFREQ]
              + sinf @ w1[NFREQ:2 * NFREQ]
              + xi * w1[2 * NFREQ][None, :]
              + b1_ref[...][i][None, :])
        hi = jnp.maximum(_ln_mxu(hi), 0.0)
        acc = acc + hi @ w2_ref[...][i]
    relh = jnp.maximum(_ln_mxu(acc), 0.0) @ wo_ref[...] + bo_ref[...]
    r = _ln_mxu(relh)
    kr = r @ wkr_ref[...]
    vr = r @ wvr_ref[...]
    ksvs = ksvs_ref[...]
    k_e = ksvs[:, :HID] + kr
    v_e = ksvs[:, HID:] + vr
    prod = qd_ref[...] * k_e
    hsel = (lax.broadcasted_iota(jnp.int32, (HID, NHEAD), 0) // HDIM
            == lax.broadcasted_iota(jnp.int32, (HID, NHEAD), 1))
    bd = hsel.astype(jnp.float32)
    sim = (prod @ bd) * SCALE
    w128 = jnp.exp(sim) @ bd.T
    u_ref[...] = w128 * v_e
    wb_ref[...] = w128
    dstv = dst_ref[0]
    dump = HN + (lax.broadcasted_iota(jnp.int32, dstv.shape, 1) % 64)
    d0 = jnp.where(dstv < HN, dstv, dump)
    d1 = jnp.where(dstv >= HN, dstv - HN, dump)
    didx_ref[...] = jnp.concatenate([d0, d1], axis=0)


def _edge_call(pl_rel, ksvs, qd, dst3, p):
    full = lambda shape: pl.BlockSpec(shape, lambda i: tuple(0 for _ in shape))
    return pl.pallas_call(
        _edge_body,
        grid=(E // T_EDGE,),
        in_specs=[
            pl.BlockSpec((T_EDGE, 3), lambda i: (i, 0)),
            pl.BlockSpec((T_EDGE, 2 * HID), lambda i: (i, 0)),
            pl.BlockSpec((T_EDGE, HID), lambda i: (i, 0)),
            pl.BlockSpec((1, 1, T_EDGE), lambda i: (i, 0, 0)),
            full((3, NFREQ)),
            full((3, 2 * NFREQ + 1, HID)),
            full((3, HID)),
            full((3, HID, HID)),
            full((HID, HID)),
            full((1, HID)),
            full((HID, HID)),
            full((HID, HID)),
        ],
        out_specs=[
            pl.BlockSpec((T_EDGE, HID), lambda i: (i, 0)),
            pl.BlockSpec((T_EDGE, HID), lambda i: (i, 0)),
            pl.BlockSpec((2, T_EDGE), lambda i: (0, i)),
        ],
        out_shape=[
            jax.ShapeDtypeStruct((E, HID), jnp.float32),
            jax.ShapeDtypeStruct((E, HID), jnp.float32),
            jax.ShapeDtypeStruct((2, E), jnp.int32),
        ],
    )(pl_rel, ksvs, qd, dst3, p['fe_freqs'], p['fe_w1'], p['fe_b1'],
      p['fe_w2'], p['fe_wo'], p['fe_bo'].reshape(1, HID), p['wkr'], p['wvr'])


def _make_scatter():
    mesh = plsc.VectorSubcoreMesh(core_axis_name="c", subcore_axis_name="s")

    @functools.partial(
        pl.kernel,
        mesh=mesh,
        out_type=jax.ShapeDtypeStruct((2, NR, HID), jnp.float32),
        scratch_types=[
            pltpu.VMEM((C_EDGE,), jnp.int32),
            pltpu.VMEM((C_EDGE,), jnp.int32),
            pltpu.VMEM((C_EDGE, HID), jnp.float32),
            pltpu.VMEM((C_EDGE, HID), jnp.float32),
            pltpu.SemaphoreType.DMA,
            pltpu.SemaphoreType.DMA,
            pltpu.VMEM_SHARED((NR, HID), jnp.float32),
        ],
    )
    def scat(vals_hbm, idx_hbm, z_hbm, out_hbm, didx0, didx1, vbuf0, vbuf1,
             sem0, sem1, acc):
        c = lax.axis_index("c")
        s = lax.axis_index("s")
        pltpu.sync_copy(z_hbm, acc.at[pl.ds(s * STRIPE, STRIPE)])
        plsc.subcore_barrier()
        per = E // 16
        nch = per // C_EDGE
        bufs = ((didx0, vbuf0, sem0), (didx1, vbuf1, sem1))

        def load(i, didx, vbuf, sem):
            base = s * per + i * C_EDGE
            pltpu.async_copy(idx_hbm.at[pl.ds(c * E + base, C_EDGE)],
                             didx, sem)
            pltpu.async_copy(vals_hbm.at[pl.ds(base, C_EDGE)], vbuf, sem)

        def drain(didx, vbuf, sem):
            pltpu.make_async_copy(idx_hbm.at[pl.ds(0, C_EDGE)],
                                  didx, sem).wait()
            pltpu.make_async_copy(vals_hbm.at[pl.ds(0, C_EDGE)],
                                  vbuf, sem).wait()

        load(0, *bufs[0])

        def body(g, carry):
            for b in range(2):
                i = 2 * g + b
                didx, vbuf, sem = bufs[b]
                ndidx, nvbuf, nsem = bufs[1 - b]

                @pl.when(i + 1 < nch)
                def _():
                    load(i + 1, ndidx, nvbuf, nsem)

                drain(didx, vbuf, sem)
                pltpu.sync_copy(vbuf, acc.at[didx], add=True)
            return carry

        lax.fori_loop(0, nch // 2, body, 0)
        plsc.subcore_barrier()
        pltpu.sync_copy(acc.at[pl.ds(s * STRIPE, STRIPE)],
                        out_hbm.at[c, pl.ds(s * STRIPE, STRIPE)])

    return scat


def _final_body(agg_ref, den_ref, xd_ref, draw_ref,
                wg_ref, bg_ref, ws_ref, bs_ref, wo_ref, bo_ref, w1_ref, b1_ref,
                w2_ref, b2_ref, out_ref):
    agg_u = agg_ref[0]
    den128 = den_ref[0]
    agg = agg_u / (den128 + 1e-9)
    xd = xd_ref[...]
    wg = wg_ref[...]
    g = jax.nn.sigmoid(agg @ wg[:HID] + xd @ wg[HID:] + bg_ref[...])
    sv = xd @ ws_ref[...] + bs_ref[...]
    outv = agg + g * (sv - agg)
    x = draw_ref[...] + outv @ wo_ref[...] + bo_ref[...]
    h = _ln(x)
    out_ref[...] = (x + jnp.maximum(h @ w1_ref[...] + b1_ref[...], 0.0)
                    @ w2_ref[...] + b2_ref[...])


def _final_call(agg2, den2, xd, d, p):
    full = lambda shape: pl.BlockSpec(shape, lambda i: tuple(0 for _ in shape))
    row = lambda w: pl.BlockSpec((T_DST, w), lambda i: (i, 0))
    nhalf = HN // T_DST
    acc_spec = pl.BlockSpec((1, T_DST, HID),
                            lambda i: (i // nhalf, i % nhalf, 0))
    return pl.pallas_call(
        _final_body,
        grid=(N_DST // T_DST,),
        in_specs=[
            acc_spec, acc_spec,
            row(HID), row(HID),
            full((2 * HID, HID)), full((1, HID)),
            full((HID, HID)), full((1, HID)),
            full((HID, HID)), full((1, HID)),
            full((HID, 4 * HID)), full((1, 4 * HID)),
            full((4 * HID, HID)), full((1, HID)),
        ],
        out_specs=pl.BlockSpec((T_DST, HID), lambda i: (i, 0)),
        out_shape=jax.ShapeDtypeStruct((N_DST, HID), jnp.float32),
    )(agg2, den2, xd, d,
      p['wg'], p['bg'].reshape(1, HID),
      p['ws'], p['bs'].reshape(1, HID),
      p['wo'], p['bo'].reshape(1, HID),
      p['w1'], p['b1'].reshape(1, 4 * HID),
      p['w2'], p['b2'].reshape(1, HID))


def kernel(map_x, edges, pl_rel, dst_x, params):
    p = params
    src = edges[0]
    dst = edges[1]
    d = dst_x.reshape(N_DST, HID)

    kv, q, xd = _node_prep(map_x, d, p)
    ksvs, qd = _make_gather()(kv, q, src, dst)
    dst3 = dst.reshape(E // T_EDGE, 1, T_EDGE)
    u, wb, didx2 = _edge_call(pl_rel, ksvs, qd, dst3, p)
    idx_cat = didx2.reshape(2 * E)
    z = jnp.zeros((STRIPE, HID), jnp.float32)
    scat = _make_scatter()
    agg2 = scat(u, idx_cat, z)
    den2 = scat(wb, idx_cat, z)
    y = _final_call(agg2, den2, xd, d, p)
    return y.reshape(dst_x.shape)
